# Initial kernel scaffold; baseline (speedup 1.0000x reference)
import jax
import jax.numpy as jnp
from jax.experimental import pallas as pl


def kernel(x, edge_index, W1, b1, W2, b2, eps1, eps2, g1a, be1a, g1b, be1b, g2a, be2a, g2b, be2b, Wu, bu):
    raise NotImplementedError("write your pallas kernel here")


# speedup vs baseline: 6.5304x; 6.5304x over previous
import functools

import jax
import jax.numpy as jnp
from jax import lax
from jax.experimental import pallas as pl
from jax.experimental.pallas import tpu as pltpu
from jax.experimental.pallas import tpu_sc as plsc

_N = 10000
_E = 320000
_H = 128

_NC = 2
_NS = 16
_NW = _NC * _NS
_C = 80
_CHUNKS = _E // _C
_CPW = _CHUNKS // _NW
_NP = 10240
_RPT = _NP // _NS


def _segsum_body(x_hbm, src_hbm, dst_hbm, zeros_hbm, out_hbm,
                 src_v, dst_v, rows_v, acc, sem):
    c = lax.axis_index("c")
    s = lax.axis_index("s")
    wid = s * _NC + c

    pltpu.sync_copy(zeros_hbm.at[pl.ds(s * _RPT, _RPT)],
                    acc.at[pl.ds(s * _RPT, _RPT)])
    pltpu.sync_copy(src_hbm.at[wid], src_v)
    pltpu.sync_copy(dst_hbm.at[wid], dst_v)
    plsc.subcore_barrier()

    def body(j, carry):
        pltpu.async_copy(x_hbm.at[src_v.at[j]], rows_v, sem).wait()
        pltpu.sync_copy(rows_v, acc.at[dst_v.at[j]], add=True)
        return carry

    lax.fori_loop(0, _CPW, body, 0)
    plsc.subcore_barrier()
    pltpu.sync_copy(acc.at[pl.ds(s * _RPT, _RPT)],
                    out_hbm.at[c, pl.ds(s * _RPT, _RPT)])


_segsum = functools.partial(
    pl.kernel,
    out_type=jax.ShapeDtypeStruct((_NC, _NP, _H), jnp.float32),
    mesh=plsc.VectorSubcoreMesh(core_axis_name="c", subcore_axis_name="s"),
    scratch_types=[
        pltpu.VMEM((_CPW, _C), jnp.int32),
        pltpu.VMEM((_CPW, _C), jnp.int32),
        pltpu.VMEM((_C, _H), jnp.float32),
        pltpu.VMEM_SHARED((_NP, _H), jnp.float32),
        pltpu.SemaphoreType.DMA,
    ],
)(_segsum_body)


def _bn_relu(z, g, b):
    mean = jnp.mean(z, axis=0, keepdims=True)
    d = z - mean
    var = jnp.mean(d * d, axis=0, keepdims=True)
    return jnp.maximum(d * lax.rsqrt(var + 1e-5) * g + b, 0.0)


def _dense1_body(h_ref, p_ref, w_ref, b_ref, eps_ref, ga_ref, ba_ref,
                 gb_ref, bb_ref, o_ref):
    msg = p_ref[0, :_N, :] + p_ref[1, :_N, :]
    hp = (1.0 + eps_ref[0, 0]) * h_ref[...] + msg
    z = jnp.dot(hp, w_ref[...], preferred_element_type=jnp.float32) + b_ref[...]
    u = _bn_relu(z, ga_ref[...], ba_ref[...])
    o_ref[...] = _bn_relu(u, gb_ref[...], bb_ref[...])


def _dense2_body(h_ref, p_ref, w_ref, b_ref, eps_ref, ga_ref, ba_ref,
                 gb_ref, bb_ref, wu_ref, bu_ref, o_ref):
    msg = p_ref[0, :_N, :] + p_ref[1, :_N, :]
    hp = (1.0 + eps_ref[0, 0]) * h_ref[...] + msg
    z = jnp.dot(hp, w_ref[...], preferred_element_type=jnp.float32) + b_ref[...]
    u = _bn_relu(z, ga_ref[...], ba_ref[...])
    v = _bn_relu(u, gb_ref[...], bb_ref[...])
    o_ref[...] = (jnp.dot(v, wu_ref[...], preferred_element_type=jnp.float32)
                  + bu_ref[0, 0])


_dense1 = pl.pallas_call(
    _dense1_body,
    out_shape=jax.ShapeDtypeStruct((_N, _H), jnp.float32),
)

_dense2 = pl.pallas_call(
    _dense2_body,
    out_shape=jax.ShapeDtypeStruct((_N, 1), jnp.float32),
)


def kernel(x, edge_index, W1, b1, W2, b2, eps1, eps2,
           g1a, be1a, g1b, be1b, g2a, be2a, g2b, be2b, Wu, bu):
    src = edge_index[0].astype(jnp.int32).reshape(_NW, _CPW, _C)
    dst = edge_index[1].astype(jnp.int32).reshape(_NW, _CPW, _C)
    zeros = jnp.zeros((_NP, _H), jnp.float32)

    row = lambda v: v.reshape(1, _H)
    sca = lambda v: v.reshape(1, 1)

    p1 = _segsum(x, src, dst, zeros)
    h1 = _dense1(x, p1, W1, row(b1), sca(eps1), row(g1a), row(be1a),
                 row(g1b), row(be1b))
    p2 = _segsum(h1, src, dst, zeros)
    scores = _dense2(h1, p2, W2, row(b2), sca(eps2), row(g2a), row(be2a),
                     row(g2b), row(be2b), Wu, sca(bu))
    return scores
